# Initial kernel scaffold; baseline (speedup 1.0000x reference)
import jax
import jax.numpy as jnp
from jax.experimental import pallas as pl


def kernel(features, neighbors_index, W1, b1, W2, b2, W3, b3, W4):
    raise NotImplementedError("write your pallas kernel here")


# speedup vs baseline: 1.0516x; 1.0516x over previous
import functools

import jax
import jax.numpy as jnp
from jax import lax
from jax.experimental import pallas as pl
from jax.experimental.pallas import tpu as pltpu
from jax.experimental.pallas import tpu_sc as plsc

_B, _N, _K, _C, _Ch = 4, 10000, 5, 256, 128
_BN = _B * _N
_NSLOT = _K + 1
_NC, _NS = 2, 16
_NW = _NC * _NS
_BNP = 40960
_NPW = _BNP // _NW
_NPC = 128
_NCHUNK = _NPW // _NPC
_RT = 2000
_NRT = _N // _RT

_PREC = jax.lax.Precision.HIGHEST


def _dot(a, b):
    return jax.lax.dot_general(a, b, (((1,), (0,)), ((), ())),
                               precision=_PREC,
                               preferred_element_type=jnp.float32)


def _mm_body(x_ref, m_ref, o_ref):
    o_ref[...] = _dot(x_ref[...], m_ref[...])


def _mm(x, m):
    r, cin = x.shape
    cout = m.shape[1]
    return pl.pallas_call(
        _mm_body,
        grid=(r // _RT,),
        in_specs=[pl.BlockSpec((_RT, cin), lambda i: (i, 0)),
                  pl.BlockSpec((cin, cout), lambda i: (0, 0))],
        out_specs=pl.BlockSpec((_RT, cout), lambda i: (i, 0)),
        out_shape=jax.ShapeDtypeStruct((r, cout), jnp.float32),
    )(x, m)


def _stats_body(x_ref, mean_ref, rstd_ref):
    x = x_ref[...]
    m = jnp.mean(x, axis=0, keepdims=True)
    v = jnp.mean((x - m) * (x - m), axis=0, keepdims=True)
    mean_ref[...] = jnp.broadcast_to(m[None], (1, 8, 128))
    rstd_ref[...] = jnp.broadcast_to(jax.lax.rsqrt(v + 1e-5)[None], (1, 8, 128))


def _stats(x, col_block):
    return pl.pallas_call(
        _stats_body,
        grid=(_B,),
        in_specs=[pl.BlockSpec((_N, _Ch), lambda b: (b, col_block))],
        out_specs=[pl.BlockSpec((1, 8, 128), lambda b: (b, 0, 0))] * 2,
        out_shape=[jax.ShapeDtypeStruct((_B, 8, 128), jnp.float32)] * 2,
    )(x)


def _norm_mm_body(x_ref, mean_ref, rstd_ref, m_ref, o_ref):
    x = (x_ref[...] - mean_ref[0, 0:1, :]) * rstd_ref[0, 0:1, :]
    x = jnp.where(x >= 0, x, 0.01 * x)
    o_ref[...] = _dot(x, m_ref[...])


def _norm_mm(x, mean, rstd, m):
    cout = m.shape[1]
    return pl.pallas_call(
        _norm_mm_body,
        grid=(_B, _NRT),
        in_specs=[pl.BlockSpec((_RT, _Ch), lambda b, j: (b * _NRT + j, 0)),
                  pl.BlockSpec((1, 8, 128), lambda b, j: (b, 0, 0)),
                  pl.BlockSpec((1, 8, 128), lambda b, j: (b, 0, 0)),
                  pl.BlockSpec((_Ch, cout), lambda b, j: (0, 0))],
        out_specs=pl.BlockSpec((_RT, cout), lambda b, j: (b * _NRT + j, 0)),
        out_shape=jax.ShapeDtypeStruct((_BN, cout), jnp.float32),
    )(x, mean, rstd, m)


def _final_body(x2_ref, r_ref, mean2_ref, rstd2_ref, meanr_ref, rstdr_ref,
                w4_ref, o_ref):
    x2n = (x2_ref[...] - mean2_ref[0, 0:1, :]) * rstd2_ref[0, 0:1, :]
    rn = (r_ref[...] - meanr_ref[0, 0:1, :]) * rstdr_ref[0, 0:1, :]
    y = x2n + rn
    y = jnp.where(y >= 0, y, 0.01 * y)
    s = jnp.sum(y * w4_ref[0:1, :], axis=1, keepdims=True)
    o_ref[...] = jnp.broadcast_to(s, (_RT, 8))


def _final(x2, p1, mean2, rstd2, meanr, rstdr, w4b):
    return pl.pallas_call(
        _final_body,
        grid=(_B, _NRT),
        in_specs=[pl.BlockSpec((_RT, _Ch), lambda b, j: (b * _NRT + j, 0)),
                  pl.BlockSpec((_RT, _Ch), lambda b, j: (b * _NRT + j, _NSLOT)),
                  pl.BlockSpec((1, 8, 128), lambda b, j: (b, 0, 0)),
                  pl.BlockSpec((1, 8, 128), lambda b, j: (b, 0, 0)),
                  pl.BlockSpec((1, 8, 128), lambda b, j: (b, 0, 0)),
                  pl.BlockSpec((1, 8, 128), lambda b, j: (b, 0, 0)),
                  pl.BlockSpec((8, 128), lambda b, j: (0, 0))],
        out_specs=pl.BlockSpec((_RT, 8), lambda b, j: (b * _NRT + j, 0)),
        out_shape=jax.ShapeDtypeStruct((_BN, 8), jnp.float32),
    )(x2, p1, mean2, rstd2, meanr, rstdr, w4b)


def _gather6_body(table, idxs, out, idx_v, rows_v, acc_v, sem):
    wid = lax.axis_index("s") * _NC + lax.axis_index("c")

    def chunk(ci, carry):
        r = wid * _NCHUNK + ci
        pltpu.sync_copy(idxs.at[r], idx_v)
        copies = [
            pltpu.async_copy(
                table.at[idx_v.at[pl.ds(k * _NPC, _NPC)]], rows_v.at[k], sem)
            for k in range(_NSLOT)
        ]
        for cp in copies:
            cp.wait()

        def node(i, c2):
            for c8 in range(_Ch // 16):
                sl = pl.ds(c8 * 16, 16)
                acc_v[i, sl] = ((rows_v[0, i, sl] + rows_v[1, i, sl])
                                + (rows_v[2, i, sl] + rows_v[3, i, sl])
                                + (rows_v[4, i, sl] + rows_v[5, i, sl]))
            return c2

        lax.fori_loop(0, _NPC, node, 0)
        pltpu.sync_copy(acc_v, out.at[pl.ds(r * _NPC, _NPC)])
        return carry

    lax.fori_loop(0, _NCHUNK, chunk, 0)


def _gather6(table, idxs):
    mesh = plsc.VectorSubcoreMesh(core_axis_name="c", subcore_axis_name="s",
                                  num_cores=_NC, num_subcores=_NS)
    run = pl.kernel(
        _gather6_body,
        out_type=jax.ShapeDtypeStruct((_BNP, _Ch), jnp.float32),
        mesh=mesh,
        scratch_types=[
            pltpu.VMEM((_NSLOT * _NPC,), jnp.int32),
            pltpu.VMEM((_NSLOT, _NPC, _Ch), jnp.float32),
            pltpu.VMEM((_NPC, _Ch), jnp.float32),
            pltpu.SemaphoreType.DMA,
        ],
    )
    return run(table, idxs)


def _mk_idx(ni, stride):
    iota = jnp.arange(_BN, dtype=jnp.int32)[:, None]
    ks = jnp.arange(_K, dtype=jnp.int32)[None, :]
    fused = jnp.concatenate([iota * stride, ni * stride + (ks + 1)], axis=1)
    fused = jnp.pad(fused, ((0, _BNP - _BN), (0, 0)))
    return (fused.reshape(_NW * _NCHUNK, _NPC, _NSLOT)
            .transpose(0, 2, 1).reshape(_NW * _NCHUNK, _NSLOT * _NPC))


def kernel(features, neighbors_index, W1, b1, W2, b2, W3, b3, W4):
    del b1, b2, b3
    flat = features.reshape(_BN, _C)
    ni = neighbors_index.reshape(_BN, _K).astype(jnp.int32)

    w1t = W1.T
    m1 = jnp.concatenate(
        [w1t[_C * j:_C * (j + 1)] for j in range(_NSLOT)] + [W3.T], axis=1)
    w2t = W2.T
    m2 = jnp.concatenate(
        [w2t[_Ch * j:_Ch * (j + 1)] for j in range(_NSLOT)], axis=1)
    w4b = jnp.broadcast_to(W4, (8, 128))

    idx1 = _mk_idx(ni, _NSLOT + 1)
    idx2 = _mk_idx(ni, _NSLOT)

    p1 = _mm(flat, m1)
    xpre1 = _gather6(p1.reshape(_BN * (_NSLOT + 1), _Ch), idx1)
    mean1, rstd1 = _stats(xpre1, 0)
    p2 = _norm_mm(xpre1, mean1, rstd1, m2)
    xpre2 = _gather6(p2.reshape(_BN * _NSLOT, _Ch), idx2)
    mean2, rstd2 = _stats(xpre2, 0)
    meanr, rstdr = _stats(p1, _NSLOT)
    out8 = _final(xpre2, p1, mean2, rstd2, meanr, rstdr, w4b)
    return out8[:, 0:1].reshape(_B, _N, 1)


# speedup vs baseline: 1.0919x; 1.0384x over previous
import functools

import jax
import jax.numpy as jnp
from jax import lax
from jax.experimental import pallas as pl
from jax.experimental.pallas import tpu as pltpu
from jax.experimental.pallas import tpu_sc as plsc

_B, _N, _K, _C, _Ch = 4, 10000, 5, 256, 128
_BN = _B * _N
_NSLOT = _K + 1
_NC, _NS = 2, 16
_NW = _NC * _NS
_BNP = 40960
_NPW = _BNP // _NW
_NPC = 64
_NCHUNK = _NPW // _NPC
_RT = 2000
_NRT = _N // _RT

_PREC = jax.lax.Precision.HIGHEST


def _dot(a, b):
    return jax.lax.dot_general(a, b, (((1,), (0,)), ((), ())),
                               precision=_PREC,
                               preferred_element_type=jnp.float32)


def _mm_body(x_ref, m_ref, o_ref):
    o_ref[...] = _dot(x_ref[...], m_ref[...])


def _mm(x, m):
    r, cin = x.shape
    cout = m.shape[1]
    return pl.pallas_call(
        _mm_body,
        grid=(r // _RT,),
        in_specs=[pl.BlockSpec((_RT, cin), lambda i: (i, 0)),
                  pl.BlockSpec((cin, cout), lambda i: (0, 0))],
        out_specs=pl.BlockSpec((_RT, cout), lambda i: (i, 0)),
        out_shape=jax.ShapeDtypeStruct((r, cout), jnp.float32),
    )(x, m)


def _stats_body(x_ref, mean_ref, rstd_ref):
    x = x_ref[...]
    m = jnp.mean(x, axis=0, keepdims=True)
    v = jnp.mean((x - m) * (x - m), axis=0, keepdims=True)
    mean_ref[...] = jnp.broadcast_to(m[None], (1, 8, 128))
    rstd_ref[...] = jnp.broadcast_to(jax.lax.rsqrt(v + 1e-5)[None], (1, 8, 128))


def _stats(x, col_block):
    return pl.pallas_call(
        _stats_body,
        grid=(_B,),
        in_specs=[pl.BlockSpec((_N, _Ch), lambda b: (b, col_block))],
        out_specs=[pl.BlockSpec((1, 8, 128), lambda b: (b, 0, 0))] * 2,
        out_shape=[jax.ShapeDtypeStruct((_B, 8, 128), jnp.float32)] * 2,
    )(x)


def _norm_mm_body(x_ref, mean_ref, rstd_ref, m_ref, o_ref):
    x = (x_ref[...] - mean_ref[0, 0:1, :]) * rstd_ref[0, 0:1, :]
    x = jnp.where(x >= 0, x, 0.01 * x)
    o_ref[...] = _dot(x, m_ref[...])


def _norm_mm(x, mean, rstd, m):
    cout = m.shape[1]
    return pl.pallas_call(
        _norm_mm_body,
        grid=(_B, _NRT),
        in_specs=[pl.BlockSpec((_RT, _Ch), lambda b, j: (b * _NRT + j, 0)),
                  pl.BlockSpec((1, 8, 128), lambda b, j: (b, 0, 0)),
                  pl.BlockSpec((1, 8, 128), lambda b, j: (b, 0, 0)),
                  pl.BlockSpec((_Ch, cout), lambda b, j: (0, 0))],
        out_specs=pl.BlockSpec((_RT, cout), lambda b, j: (b * _NRT + j, 0)),
        out_shape=jax.ShapeDtypeStruct((_BN, cout), jnp.float32),
    )(x, mean, rstd, m)


def _final_body(x2_ref, r_ref, mean2_ref, rstd2_ref, meanr_ref, rstdr_ref,
                w4_ref, o_ref):
    x2n = (x2_ref[...] - mean2_ref[0, 0:1, :]) * rstd2_ref[0, 0:1, :]
    rn = (r_ref[...] - meanr_ref[0, 0:1, :]) * rstdr_ref[0, 0:1, :]
    y = x2n + rn
    y = jnp.where(y >= 0, y, 0.01 * y)
    s = jnp.sum(y * w4_ref[0:1, :], axis=1, keepdims=True)
    o_ref[...] = jnp.broadcast_to(s, (_RT, 8))


def _final(x2, p1, mean2, rstd2, meanr, rstdr, w4b):
    return pl.pallas_call(
        _final_body,
        grid=(_B, _NRT),
        in_specs=[pl.BlockSpec((_RT, _Ch), lambda b, j: (b * _NRT + j, 0)),
                  pl.BlockSpec((_RT, _Ch), lambda b, j: (b * _NRT + j, _NSLOT)),
                  pl.BlockSpec((1, 8, 128), lambda b, j: (b, 0, 0)),
                  pl.BlockSpec((1, 8, 128), lambda b, j: (b, 0, 0)),
                  pl.BlockSpec((1, 8, 128), lambda b, j: (b, 0, 0)),
                  pl.BlockSpec((1, 8, 128), lambda b, j: (b, 0, 0)),
                  pl.BlockSpec((8, 128), lambda b, j: (0, 0))],
        out_specs=pl.BlockSpec((_RT, 8), lambda b, j: (b * _NRT + j, 0)),
        out_shape=jax.ShapeDtypeStruct((_BN, 8), jnp.float32),
    )(x2, p1, mean2, rstd2, meanr, rstdr, w4b)


def _gather6_body(table, idxs, out, idx_v, rows_v, acc_v, sem):
    wid = lax.axis_index("s") * _NC + lax.axis_index("c")
    base = wid * _NCHUNK

    def gathers(ci, buf):
        return [
            pltpu.make_async_copy(
                table.at[idx_v.at[buf].at[pl.ds(k * _NPC, _NPC)]],
                rows_v.at[buf].at[k], sem)
            for k in range(_NSLOT)
        ]

    def fire(ci, buf):
        pltpu.sync_copy(idxs.at[base + ci], idx_v.at[buf])
        for cp in gathers(ci, buf):
            cp.start()

    def chunk(ci, carry):
        p = lax.rem(ci, 2)

        @pl.when(ci < _NCHUNK - 1)
        def _():
            fire(ci + 1, 1 - p)

        for cp in gathers(ci, p):
            cp.wait()

        def node(i, c2):
            for c8 in range(_Ch // 16):
                sl = pl.ds(c8 * 16, 16)
                acc_v[i, sl] = ((rows_v[p, 0, i, sl] + rows_v[p, 1, i, sl])
                                + (rows_v[p, 2, i, sl] + rows_v[p, 3, i, sl])
                                + (rows_v[p, 4, i, sl] + rows_v[p, 5, i, sl]))
            return c2

        lax.fori_loop(0, _NPC, node, 0)
        pltpu.sync_copy(acc_v, out.at[pl.ds((base + ci) * _NPC, _NPC)])
        return carry

    fire(0, 0)
    lax.fori_loop(0, _NCHUNK, chunk, 0)


def _gather6(table, idxs):
    mesh = plsc.VectorSubcoreMesh(core_axis_name="c", subcore_axis_name="s",
                                  num_cores=_NC, num_subcores=_NS)
    run = pl.kernel(
        _gather6_body,
        out_type=jax.ShapeDtypeStruct((_BNP, _Ch), jnp.float32),
        mesh=mesh,
        scratch_types=[
            pltpu.VMEM((2, _NSLOT * _NPC), jnp.int32),
            pltpu.VMEM((2, _NSLOT, _NPC, _Ch), jnp.float32),
            pltpu.VMEM((_NPC, _Ch), jnp.float32),
            pltpu.SemaphoreType.DMA,
        ],
    )
    return run(table, idxs)


def _mk_idx(ni, stride):
    iota = jnp.arange(_BN, dtype=jnp.int32)[:, None]
    ks = jnp.arange(_K, dtype=jnp.int32)[None, :]
    fused = jnp.concatenate([iota * stride, ni * stride + (ks + 1)], axis=1)
    fused = jnp.pad(fused, ((0, _BNP - _BN), (0, 0)))
    return (fused.reshape(_NW * _NCHUNK, _NPC, _NSLOT)
            .transpose(0, 2, 1).reshape(_NW * _NCHUNK, _NSLOT * _NPC))


def kernel(features, neighbors_index, W1, b1, W2, b2, W3, b3, W4):
    del b1, b2, b3
    flat = features.reshape(_BN, _C)
    ni = neighbors_index.reshape(_BN, _K).astype(jnp.int32)

    w1t = W1.T
    m1 = jnp.concatenate(
        [w1t[_C * j:_C * (j + 1)] for j in range(_NSLOT)] + [W3.T], axis=1)
    w2t = W2.T
    m2 = jnp.concatenate(
        [w2t[_Ch * j:_Ch * (j + 1)] for j in range(_NSLOT)], axis=1)
    w4b = jnp.broadcast_to(W4, (8, 128))

    idx1 = _mk_idx(ni, _NSLOT + 1)
    idx2 = _mk_idx(ni, _NSLOT)

    p1 = _mm(flat, m1)
    xpre1 = _gather6(p1.reshape(_BN * (_NSLOT + 1), _Ch), idx1)
    mean1, rstd1 = _stats(xpre1, 0)
    p2 = _norm_mm(xpre1, mean1, rstd1, m2)
    xpre2 = _gather6(p2.reshape(_BN * _NSLOT, _Ch), idx2)
    mean2, rstd2 = _stats(xpre2, 0)
    meanr, rstdr = _stats(p1, _NSLOT)
    out8 = _final(xpre2, p1, mean2, rstd2, meanr, rstdr, w4b)
    return out8[:, 0:1].reshape(_B, _N, 1)


# speedup vs baseline: 1.2817x; 1.1738x over previous
import functools

import jax
import jax.numpy as jnp
from jax import lax
from jax.experimental import pallas as pl
from jax.experimental.pallas import tpu as pltpu
from jax.experimental.pallas import tpu_sc as plsc

_B, _N, _K, _C, _Ch = 4, 10000, 5, 256, 128
_BN = _B * _N
_NSLOT = _K + 1
_NC, _NS = 2, 16
_NW = _NC * _NS
_BNP = 40960
_NPC = 64
_TOTCH = _BNP // _NPC
_NCH0 = 30
_NCH1 = (_TOTCH - _NS * _NCH0) // _NS
_RT = 2000
_NRT = _N // _RT

_PREC = jax.lax.Precision.HIGHEST


def _dot(a, b):
    return jax.lax.dot_general(a, b, (((1,), (0,)), ((), ())),
                               precision=_PREC,
                               preferred_element_type=jnp.float32)


def _lrelu(x):
    return jnp.where(x >= 0, x, 0.01 * x)


def _mm_body(x_ref, m_ref, o_ref):
    o_ref[...] = _dot(x_ref[...], m_ref[...])


def _mm(x, m):
    r, cin = x.shape
    cout = m.shape[1]
    return pl.pallas_call(
        _mm_body,
        grid=(r // _RT,),
        in_specs=[pl.BlockSpec((_RT, cin), lambda i: (i, 0)),
                  pl.BlockSpec((cin, cout), lambda i: (0, 0))],
        out_specs=pl.BlockSpec((_RT, cout), lambda i: (i, 0)),
        out_shape=jax.ShapeDtypeStruct((r, cout), jnp.float32),
    )(x, m)


def _stats2_body(xp_ref, self_ref, mean_ref, rstd_ref):
    x = xp_ref[...] + self_ref[...]
    m = jnp.mean(x, axis=0, keepdims=True)
    v = jnp.mean((x - m) * (x - m), axis=0, keepdims=True)
    mean_ref[...] = jnp.broadcast_to(m[None], (1, 8, 128))
    rstd_ref[...] = jnp.broadcast_to(jax.lax.rsqrt(v + 1e-5)[None], (1, 8, 128))


def _stats2(xp, p, col_block):
    return pl.pallas_call(
        _stats2_body,
        grid=(_B,),
        in_specs=[pl.BlockSpec((_N, _Ch), lambda b: (b, 0)),
                  pl.BlockSpec((_N, _Ch), lambda b: (b, col_block))],
        out_specs=[pl.BlockSpec((1, 8, 128), lambda b: (b, 0, 0))] * 2,
        out_shape=[jax.ShapeDtypeStruct((_B, 8, 128), jnp.float32)] * 2,
    )(xp, p)


def _stats_body(x_ref, mean_ref, rstd_ref):
    x = x_ref[...]
    m = jnp.mean(x, axis=0, keepdims=True)
    v = jnp.mean((x - m) * (x - m), axis=0, keepdims=True)
    mean_ref[...] = jnp.broadcast_to(m[None], (1, 8, 128))
    rstd_ref[...] = jnp.broadcast_to(jax.lax.rsqrt(v + 1e-5)[None], (1, 8, 128))


def _stats(x, col_block):
    return pl.pallas_call(
        _stats_body,
        grid=(_B,),
        in_specs=[pl.BlockSpec((_N, _Ch), lambda b: (b, col_block))],
        out_specs=[pl.BlockSpec((1, 8, 128), lambda b: (b, 0, 0))] * 2,
        out_shape=[jax.ShapeDtypeStruct((_B, 8, 128), jnp.float32)] * 2,
    )(x)


def _norm_mm_body(xp_ref, self_ref, mean_ref, rstd_ref, m_ref, o_ref):
    x = (xp_ref[...] + self_ref[...] - mean_ref[0, 0:1, :]) * rstd_ref[0, 0:1, :]
    o_ref[...] = _dot(_lrelu(x), m_ref[...])


def _norm_mm(xp, p, mean, rstd, m):
    cout = m.shape[1]
    return pl.pallas_call(
        _norm_mm_body,
        grid=(_B, _NRT),
        in_specs=[pl.BlockSpec((_RT, _Ch), lambda b, j: (b * _NRT + j, 0)),
                  pl.BlockSpec((_RT, _Ch), lambda b, j: (b * _NRT + j, 0)),
                  pl.BlockSpec((1, 8, 128), lambda b, j: (b, 0, 0)),
                  pl.BlockSpec((1, 8, 128), lambda b, j: (b, 0, 0)),
                  pl.BlockSpec((_Ch, cout), lambda b, j: (0, 0))],
        out_specs=pl.BlockSpec((_RT, cout), lambda b, j: (b * _NRT + j, 0)),
        out_shape=jax.ShapeDtypeStruct((_BN, cout), jnp.float32),
    )(xp, p, mean, rstd, m)


def _final_body(x2_ref, self2_ref, r_ref, mean2_ref, rstd2_ref, meanr_ref,
                rstdr_ref, w4_ref, o_ref):
    x2n = ((x2_ref[...] + self2_ref[...] - mean2_ref[0, 0:1, :])
           * rstd2_ref[0, 0:1, :])
    rn = (r_ref[...] - meanr_ref[0, 0:1, :]) * rstdr_ref[0, 0:1, :]
    y = _lrelu(x2n + rn)
    s = jnp.sum(y * w4_ref[0:1, :], axis=1, keepdims=True)
    o_ref[...] = jnp.broadcast_to(s, (_RT, 8))


def _final(x2, p2, p1, mean2, rstd2, meanr, rstdr, w4b):
    return pl.pallas_call(
        _final_body,
        grid=(_B, _NRT),
        in_specs=[pl.BlockSpec((_RT, _Ch), lambda b, j: (b * _NRT + j, 0)),
                  pl.BlockSpec((_RT, _Ch), lambda b, j: (b * _NRT + j, 0)),
                  pl.BlockSpec((_RT, _Ch), lambda b, j: (b * _NRT + j, _NSLOT)),
                  pl.BlockSpec((1, 8, 128), lambda b, j: (b, 0, 0)),
                  pl.BlockSpec((1, 8, 128), lambda b, j: (b, 0, 0)),
                  pl.BlockSpec((1, 8, 128), lambda b, j: (b, 0, 0)),
                  pl.BlockSpec((1, 8, 128), lambda b, j: (b, 0, 0)),
                  pl.BlockSpec((8, 128), lambda b, j: (0, 0))],
        out_specs=pl.BlockSpec((_RT, 8), lambda b, j: (b * _NRT + j, 0)),
        out_shape=jax.ShapeDtypeStruct((_BN, 8), jnp.float32),
    )(x2, p2, p1, mean2, rstd2, meanr, rstdr, w4b)


def _gather5_body(table, idxs, out, idx_v, rows_v, acc_v, sem):
    cid = lax.axis_index("c")
    sid = lax.axis_index("s")
    start = jnp.where(cid == 0, sid * _NCH0, _NS * _NCH0 + sid * _NCH1)
    cnt = jnp.where(cid == 0, _NCH0, _NCH1)

    def gathers(buf):
        return [
            pltpu.make_async_copy(
                table.at[idx_v.at[buf].at[pl.ds(k * _NPC, _NPC)]],
                rows_v.at[buf].at[k], sem)
            for k in range(_K)
        ]

    def fire(ci, buf):
        pltpu.sync_copy(idxs.at[start + ci], idx_v.at[buf])
        for cp in gathers(buf):
            cp.start()

    def chunk(ci, carry):
        p = lax.rem(ci, 2)

        @pl.when(ci < cnt - 1)
        def _():
            fire(ci + 1, 1 - p)

        for cp in gathers(p):
            cp.wait()

        def node(i, c2):
            for c8 in range(_Ch // 16):
                sl = pl.ds(c8 * 16, 16)
                acc_v[i, sl] = ((rows_v[p, 0, i, sl] + rows_v[p, 1, i, sl])
                                + (rows_v[p, 2, i, sl] + rows_v[p, 3, i, sl])
                                + rows_v[p, 4, i, sl])
            return c2

        lax.fori_loop(0, _NPC, node, 0)
        pltpu.sync_copy(acc_v, out.at[pl.ds((start + ci) * _NPC, _NPC)])
        return carry

    fire(0, 0)
    lax.fori_loop(0, cnt, chunk, 0)


def _gather5(table, idxs):
    mesh = plsc.VectorSubcoreMesh(core_axis_name="c", subcore_axis_name="s",
                                  num_cores=_NC, num_subcores=_NS)
    run = pl.kernel(
        _gather5_body,
        out_type=jax.ShapeDtypeStruct((_BNP, _Ch), jnp.float32),
        mesh=mesh,
        scratch_types=[
            pltpu.VMEM((2, _K * _NPC), jnp.int32),
            pltpu.VMEM((2, _K, _NPC, _Ch), jnp.float32),
            pltpu.VMEM((_NPC, _Ch), jnp.float32),
            pltpu.SemaphoreType.DMA,
        ],
    )
    return run(table, idxs)


def _mk_idx(ni, stride):
    ks = jnp.arange(_K, dtype=jnp.int32)[None, :]
    fused = ni * stride + (ks + 1)
    fused = jnp.pad(fused, ((0, _BNP - _BN), (0, 0)))
    return (fused.reshape(_TOTCH, _NPC, _K)
            .transpose(0, 2, 1).reshape(_TOTCH, _K * _NPC))


def kernel(features, neighbors_index, W1, b1, W2, b2, W3, b3, W4):
    del b1, b2, b3
    flat = features.reshape(_BN, _C)
    ni = neighbors_index.reshape(_BN, _K).astype(jnp.int32)

    w1t = W1.T
    m1 = jnp.concatenate(
        [w1t[_C * j:_C * (j + 1)] for j in range(_NSLOT)] + [W3.T], axis=1)
    w2t = W2.T
    m2 = jnp.concatenate(
        [w2t[_Ch * j:_Ch * (j + 1)] for j in range(_NSLOT)], axis=1)
    w4b = jnp.broadcast_to(W4, (8, 128))

    idx1 = _mk_idx(ni, _NSLOT + 1)
    idx2 = _mk_idx(ni, _NSLOT)

    p1 = _mm(flat, m1)
    xpre1 = _gather5(p1.reshape(_BN * (_NSLOT + 1), _Ch), idx1)
    mean1, rstd1 = _stats2(xpre1, p1, 0)
    p2 = _norm_mm(xpre1, p1, mean1, rstd1, m2)
    xpre2 = _gather5(p2.reshape(_BN * _NSLOT, _Ch), idx2)
    mean2, rstd2 = _stats2(xpre2, p2, 0)
    meanr, rstdr = _stats(p1, _NSLOT)
    out8 = _final(xpre2, p2, p1, mean2, rstd2, meanr, rstdr, w4b)
    return out8[:, 0:1].reshape(_B, _N, 1)


# speedup vs baseline: 1.7824x; 1.3906x over previous
import functools

import jax
import jax.numpy as jnp
from jax import lax
from jax.experimental import pallas as pl
from jax.experimental.pallas import tpu as pltpu
from jax.experimental.pallas import tpu_sc as plsc

_B, _N, _K, _C, _Ch = 4, 10000, 5, 256, 128
_BN = _B * _N
_NSLOT = _K + 1
_NC, _NS = 2, 16
_NW = _NC * _NS
_BNP = 40960
_NPC = 64
_TOTCH = _BNP // _NPC
_NCH0 = 20
_NCH1 = (_TOTCH - _NS * _NCH0) // _NS
_RT = 2000
_NRT = _N // _RT

_PREC = jax.lax.Precision.HIGHEST


def _dot(a, b):
    return jax.lax.dot_general(a, b, (((1,), (0,)), ((), ())),
                               precision=_PREC,
                               preferred_element_type=jnp.float32)


def _lrelu(x):
    return jnp.where(x >= 0, x, 0.01 * x)


def _mm_body(x_ref, m_ref, o_ref):
    o_ref[...] = _dot(x_ref[...], m_ref[...])


def _mm(x, m):
    r, cin = x.shape
    cout = m.shape[1]
    return pl.pallas_call(
        _mm_body,
        grid=(r // _RT,),
        in_specs=[pl.BlockSpec((_RT, cin), lambda i: (i, 0)),
                  pl.BlockSpec((cin, cout), lambda i: (0, 0))],
        out_specs=pl.BlockSpec((_RT, cout), lambda i: (i, 0)),
        out_shape=jax.ShapeDtypeStruct((r, cout), jnp.float32),
    )(x, m)


def _stats2_body(xp_ref, self_ref, mean_ref, rstd_ref):
    x = xp_ref[...] + self_ref[...]
    m = jnp.mean(x, axis=0, keepdims=True)
    v = jnp.mean((x - m) * (x - m), axis=0, keepdims=True)
    mean_ref[...] = jnp.broadcast_to(m[None], (1, 8, 128))
    rstd_ref[...] = jnp.broadcast_to(jax.lax.rsqrt(v + 1e-5)[None], (1, 8, 128))


def _stats2(xp, p, col_block):
    return pl.pallas_call(
        _stats2_body,
        grid=(_B,),
        in_specs=[pl.BlockSpec((_N, _Ch), lambda b: (b, 0)),
                  pl.BlockSpec((_N, _Ch), lambda b: (b, col_block))],
        out_specs=[pl.BlockSpec((1, 8, 128), lambda b: (b, 0, 0))] * 2,
        out_shape=[jax.ShapeDtypeStruct((_B, 8, 128), jnp.float32)] * 2,
    )(xp, p)


def _stats_body(x_ref, mean_ref, rstd_ref):
    x = x_ref[...]
    m = jnp.mean(x, axis=0, keepdims=True)
    v = jnp.mean((x - m) * (x - m), axis=0, keepdims=True)
    mean_ref[...] = jnp.broadcast_to(m[None], (1, 8, 128))
    rstd_ref[...] = jnp.broadcast_to(jax.lax.rsqrt(v + 1e-5)[None], (1, 8, 128))


def _stats(x, col_block):
    return pl.pallas_call(
        _stats_body,
        grid=(_B,),
        in_specs=[pl.BlockSpec((_N, _Ch), lambda b: (b, col_block))],
        out_specs=[pl.BlockSpec((1, 8, 128), lambda b: (b, 0, 0))] * 2,
        out_shape=[jax.ShapeDtypeStruct((_B, 8, 128), jnp.float32)] * 2,
    )(x)


def _norm_mm_body(xp_ref, self_ref, mean_ref, rstd_ref, m_ref, o_ref):
    x = (xp_ref[...] + self_ref[...] - mean_ref[0, 0:1, :]) * rstd_ref[0, 0:1, :]
    o_ref[...] = _dot(_lrelu(x), m_ref[...])


def _norm_mm(xp, p, mean, rstd, m):
    cout = m.shape[1]
    return pl.pallas_call(
        _norm_mm_body,
        grid=(_B, _NRT),
        in_specs=[pl.BlockSpec((_RT, _Ch), lambda b, j: (b * _NRT + j, 0)),
                  pl.BlockSpec((_RT, _Ch), lambda b, j: (b * _NRT + j, 0)),
                  pl.BlockSpec((1, 8, 128), lambda b, j: (b, 0, 0)),
                  pl.BlockSpec((1, 8, 128), lambda b, j: (b, 0, 0)),
                  pl.BlockSpec((_Ch, cout), lambda b, j: (0, 0))],
        out_specs=pl.BlockSpec((_RT, cout), lambda b, j: (b * _NRT + j, 0)),
        out_shape=jax.ShapeDtypeStruct((_BN, cout), jnp.float32),
    )(xp, p, mean, rstd, m)


def _final_body(x2_ref, self2_ref, r_ref, mean2_ref, rstd2_ref, meanr_ref,
                rstdr_ref, w4_ref, o_ref):
    x2n = ((x2_ref[...] + self2_ref[...] - mean2_ref[0, 0:1, :])
           * rstd2_ref[0, 0:1, :])
    rn = (r_ref[...] - meanr_ref[0, 0:1, :]) * rstdr_ref[0, 0:1, :]
    y = _lrelu(x2n + rn)
    s = jnp.sum(y * w4_ref[0:1, :], axis=1, keepdims=True)
    o_ref[...] = jnp.broadcast_to(s, (_RT, 8))


def _final(x2, p2, p1, mean2, rstd2, meanr, rstdr, w4b):
    return pl.pallas_call(
        _final_body,
        grid=(_B, _NRT),
        in_specs=[pl.BlockSpec((_RT, _Ch), lambda b, j: (b * _NRT + j, 0)),
                  pl.BlockSpec((_RT, _Ch), lambda b, j: (b * _NRT + j, 0)),
                  pl.BlockSpec((_RT, _Ch), lambda b, j: (b * _NRT + j, _NSLOT)),
                  pl.BlockSpec((1, 8, 128), lambda b, j: (b, 0, 0)),
                  pl.BlockSpec((1, 8, 128), lambda b, j: (b, 0, 0)),
                  pl.BlockSpec((1, 8, 128), lambda b, j: (b, 0, 0)),
                  pl.BlockSpec((1, 8, 128), lambda b, j: (b, 0, 0)),
                  pl.BlockSpec((8, 128), lambda b, j: (0, 0))],
        out_specs=pl.BlockSpec((_RT, 8), lambda b, j: (b * _NRT + j, 0)),
        out_shape=jax.ShapeDtypeStruct((_BN, 8), jnp.float32),
    )(x2, p2, p1, mean2, rstd2, meanr, rstdr, w4b)


def _gather5_body(table, idxs, out, idx_v, rows_v, acc_v, sem):
    cid = lax.axis_index("c")
    sid = lax.axis_index("s")
    start = jnp.where(cid == 0, sid * _NCH0, _NS * _NCH0 + sid * _NCH1)
    cnt = jnp.where(cid == 0, _NCH0, _NCH1)

    def gathers(buf):
        return [
            pltpu.make_async_copy(
                table.at[idx_v.at[buf].at[pl.ds(k * _NPC, _NPC)]],
                rows_v.at[buf].at[k], sem)
            for k in range(_K)
        ]

    def fire(ci, buf):
        pltpu.sync_copy(idxs.at[start + ci], idx_v.at[buf])
        for cp in gathers(buf):
            cp.start()

    def chunk(ci, carry):
        p = lax.rem(ci, 2)

        @pl.when(ci < cnt - 1)
        def _():
            fire(ci + 1, 1 - p)

        for cp in gathers(p):
            cp.wait()

        def node(i, c2):
            for c8 in range(_Ch // 16):
                sl = pl.ds(c8 * 16, 16)
                acc_v[i, sl] = ((rows_v[p, 0, i, sl] + rows_v[p, 1, i, sl])
                                + (rows_v[p, 2, i, sl] + rows_v[p, 3, i, sl])
                                + rows_v[p, 4, i, sl])
            return c2

        lax.fori_loop(0, _NPC, node, 0)
        pltpu.sync_copy(acc_v, out.at[pl.ds((start + ci) * _NPC, _NPC)])
        return carry

    fire(0, 0)
    lax.fori_loop(0, cnt, chunk, 0)


def _gather5(table, idxs):
    mesh = plsc.VectorSubcoreMesh(core_axis_name="c", subcore_axis_name="s",
                                  num_cores=_NC, num_subcores=_NS)
    run = pl.kernel(
        _gather5_body,
        out_type=jax.ShapeDtypeStruct((_BNP, _Ch), jnp.float32),
        mesh=mesh,
        scratch_types=[
            pltpu.VMEM((2, _K * _NPC), jnp.int32),
            pltpu.VMEM((2, _K, _NPC, _Ch), jnp.float32),
            pltpu.VMEM((_NPC, _Ch), jnp.float32),
            pltpu.SemaphoreType.DMA,
        ],
    )
    return run(table, idxs)


def _mk_idx(ni, stride):
    ks = jnp.arange(_K, dtype=jnp.int32)[None, :]
    fused = ni * stride + (ks + 1)
    npad = _BNP - _BN
    pad_rows = (jnp.arange(npad * _K, dtype=jnp.int32).reshape(npad, _K)
                * 79) % _BN
    fused = jnp.concatenate([fused, pad_rows * stride + (ks + 1)], axis=0)
    return (fused.reshape(_TOTCH, _NPC, _K)
            .transpose(0, 2, 1).reshape(_TOTCH, _K * _NPC))


def kernel(features, neighbors_index, W1, b1, W2, b2, W3, b3, W4):
    del b1, b2, b3
    flat = features.reshape(_BN, _C)
    ni = neighbors_index.reshape(_BN, _K).astype(jnp.int32)

    w1t = W1.T
    m1 = jnp.concatenate(
        [w1t[_C * j:_C * (j + 1)] for j in range(_NSLOT)] + [W3.T], axis=1)
    w2t = W2.T
    m2 = jnp.concatenate(
        [w2t[_Ch * j:_Ch * (j + 1)] for j in range(_NSLOT)], axis=1)
    w4b = jnp.broadcast_to(W4, (8, 128))

    idx1 = _mk_idx(ni, _NSLOT + 1)
    idx2 = _mk_idx(ni, _NSLOT)

    p1 = _mm(flat, m1)
    xpre1 = _gather5(p1.reshape(_BN * (_NSLOT + 1), _Ch), idx1)
    mean1, rstd1 = _stats2(xpre1, p1, 0)
    p2 = _norm_mm(xpre1, p1, mean1, rstd1, m2)
    xpre2 = _gather5(p2.reshape(_BN * _NSLOT, _Ch), idx2)
    mean2, rstd2 = _stats2(xpre2, p2, 0)
    meanr, rstdr = _stats(p1, _NSLOT)
    out8 = _final(xpre2, p2, p1, mean2, rstd2, meanr, rstdr, w4b)
    return out8[:, 0:1].reshape(_B, _N, 1)


# speedup vs baseline: 1.9553x; 1.0970x over previous
import functools

import jax
import jax.numpy as jnp
from jax import lax
from jax.experimental import pallas as pl
from jax.experimental.pallas import tpu as pltpu
from jax.experimental.pallas import tpu_sc as plsc

_B, _N, _K, _C, _Ch = 4, 10000, 5, 256, 128
_BN = _B * _N
_NSLOT = _K + 1
_NC, _NS = 2, 16
_NW = _NC * _NS
_BNP = 40960
_NPC = 64
_TOTCH = _BNP // _NPC
_NCH0 = 20
_NCH1 = (_TOTCH - _NS * _NCH0) // _NS
_RT = 2000
_NRT = _N // _RT
_NRTG = _BN // _RT

_PREC = jax.lax.Precision.HIGHEST


def _dot(a, b):
    return jax.lax.dot_general(a, b, (((1,), (0,)), ((), ())),
                               precision=_PREC,
                               preferred_element_type=jnp.float32)


def _lrelu(x):
    return jnp.where(x >= 0, x, 0.01 * x)


def _mm_body(x_ref, m_ref, o_ref):
    o_ref[...] = _dot(x_ref[...], m_ref[...])


def _mm_slots(x, m):
    cin = x.shape[1]
    nslots = m.shape[1] // _Ch
    return pl.pallas_call(
        _mm_body,
        grid=(_NRTG, nslots),
        in_specs=[pl.BlockSpec((_RT, cin), lambda i, j: (i, 0)),
                  pl.BlockSpec((cin, _Ch), lambda i, j: (0, j))],
        out_specs=pl.BlockSpec((_RT, _Ch), lambda i, j: (j * _NRTG + i, 0)),
        out_shape=jax.ShapeDtypeStruct((nslots * _BN, _Ch), jnp.float32),
    )(x, m)


def _stats2_body(xp_ref, self_ref, mean_ref, rstd_ref):
    x = xp_ref[...] + self_ref[...]
    m = jnp.mean(x, axis=0, keepdims=True)
    v = jnp.mean((x - m) * (x - m), axis=0, keepdims=True)
    mean_ref[...] = jnp.broadcast_to(m[None], (1, 8, 128))
    rstd_ref[...] = jnp.broadcast_to(jax.lax.rsqrt(v + 1e-5)[None], (1, 8, 128))


def _stats2(xp, table):
    return pl.pallas_call(
        _stats2_body,
        grid=(_B,),
        in_specs=[pl.BlockSpec((_N, _Ch), lambda b: (b, 0)),
                  pl.BlockSpec((_N, _Ch), lambda b: (b, 0))],
        out_specs=[pl.BlockSpec((1, 8, 128), lambda b: (b, 0, 0))] * 2,
        out_shape=[jax.ShapeDtypeStruct((_B, 8, 128), jnp.float32)] * 2,
    )(xp, table)


def _stats_body(x_ref, mean_ref, rstd_ref):
    x = x_ref[...]
    m = jnp.mean(x, axis=0, keepdims=True)
    v = jnp.mean((x - m) * (x - m), axis=0, keepdims=True)
    mean_ref[...] = jnp.broadcast_to(m[None], (1, 8, 128))
    rstd_ref[...] = jnp.broadcast_to(jax.lax.rsqrt(v + 1e-5)[None], (1, 8, 128))


def _stats_slot(table, slot):
    nb = slot * (_BN // _N)
    return pl.pallas_call(
        _stats_body,
        grid=(_B,),
        in_specs=[pl.BlockSpec((_N, _Ch), lambda b: (nb + b, 0))],
        out_specs=[pl.BlockSpec((1, 8, 128), lambda b: (b, 0, 0))] * 2,
        out_shape=[jax.ShapeDtypeStruct((_B, 8, 128), jnp.float32)] * 2,
    )(table)


def _norm_mm_body(xp_ref, self_ref, mean_ref, rstd_ref, m_ref, o_ref):
    x = (xp_ref[...] + self_ref[...] - mean_ref[0, 0:1, :]) * rstd_ref[0, 0:1, :]
    o_ref[...] = _dot(_lrelu(x), m_ref[...])


def _norm_mm_slots(xp, table, mean, rstd, m):
    nslots = m.shape[1] // _Ch
    return pl.pallas_call(
        _norm_mm_body,
        grid=(_NRTG, nslots),
        in_specs=[pl.BlockSpec((_RT, _Ch), lambda i, j: (i, 0)),
                  pl.BlockSpec((_RT, _Ch), lambda i, j: (i, 0)),
                  pl.BlockSpec((1, 8, 128), lambda i, j: (i // _NRT, 0, 0)),
                  pl.BlockSpec((1, 8, 128), lambda i, j: (i // _NRT, 0, 0)),
                  pl.BlockSpec((_Ch, _Ch), lambda i, j: (0, j))],
        out_specs=pl.BlockSpec((_RT, _Ch), lambda i, j: (j * _NRTG + i, 0)),
        out_shape=jax.ShapeDtypeStruct((nslots * _BN, _Ch), jnp.float32),
    )(xp, table, mean, rstd, m)


def _final_body(x2_ref, self2_ref, r_ref, mean2_ref, rstd2_ref, meanr_ref,
                rstdr_ref, w4_ref, o_ref):
    x2n = ((x2_ref[...] + self2_ref[...] - mean2_ref[0, 0:1, :])
           * rstd2_ref[0, 0:1, :])
    rn = (r_ref[...] - meanr_ref[0, 0:1, :]) * rstdr_ref[0, 0:1, :]
    y = _lrelu(x2n + rn)
    s = jnp.sum(y * w4_ref[0:1, :], axis=1, keepdims=True)
    o_ref[...] = jnp.broadcast_to(s, (_RT, 8))


def _final(x2, table2, table1, mean2, rstd2, meanr, rstdr, w4b):
    rslot = _NSLOT * _NRTG
    return pl.pallas_call(
        _final_body,
        grid=(_NRTG,),
        in_specs=[pl.BlockSpec((_RT, _Ch), lambda i: (i, 0)),
                  pl.BlockSpec((_RT, _Ch), lambda i: (i, 0)),
                  pl.BlockSpec((_RT, _Ch), lambda i: (rslot + i, 0)),
                  pl.BlockSpec((1, 8, 128), lambda i: (i // _NRT, 0, 0)),
                  pl.BlockSpec((1, 8, 128), lambda i: (i // _NRT, 0, 0)),
                  pl.BlockSpec((1, 8, 128), lambda i: (i // _NRT, 0, 0)),
                  pl.BlockSpec((1, 8, 128), lambda i: (i // _NRT, 0, 0)),
                  pl.BlockSpec((8, 128), lambda i: (0, 0))],
        out_specs=pl.BlockSpec((_RT, 8), lambda i: (i, 0)),
        out_shape=jax.ShapeDtypeStruct((_BN, 8), jnp.float32),
    )(x2, table2, table1, mean2, rstd2, meanr, rstdr, w4b)


def _gather5_body(table, idxs, out, idx_v, rows_v, acc_v, sem):
    cid = lax.axis_index("c")
    sid = lax.axis_index("s")
    start = jnp.where(cid == 0, sid * _NCH0, _NS * _NCH0 + sid * _NCH1)
    cnt = jnp.where(cid == 0, _NCH0, _NCH1)

    def gathers(buf):
        return [
            pltpu.make_async_copy(
                table.at[idx_v.at[buf].at[pl.ds(k * _NPC, _NPC)]],
                rows_v.at[buf].at[k], sem)
            for k in range(_K)
        ]

    def fire(ci, buf):
        pltpu.sync_copy(idxs.at[start + ci], idx_v.at[buf])
        for cp in gathers(buf):
            cp.start()

    def chunk(ci, carry):
        p = lax.rem(ci, 2)

        @pl.when(ci < cnt - 1)
        def _():
            fire(ci + 1, 1 - p)

        for cp in gathers(p):
            cp.wait()

        def node(i, c2):
            for c8 in range(_Ch // 16):
                sl = pl.ds(c8 * 16, 16)
                acc_v[i, sl] = ((rows_v[p, 0, i, sl] + rows_v[p, 1, i, sl])
                                + (rows_v[p, 2, i, sl] + rows_v[p, 3, i, sl])
                                + rows_v[p, 4, i, sl])
            return c2

        lax.fori_loop(0, _NPC, node, 0)
        pltpu.sync_copy(acc_v, out.at[pl.ds((start + ci) * _NPC, _NPC)])
        return carry

    fire(0, 0)
    lax.fori_loop(0, cnt, chunk, 0)


def _gather5(table, idxs):
    mesh = plsc.VectorSubcoreMesh(core_axis_name="c", subcore_axis_name="s",
                                  num_cores=_NC, num_subcores=_NS)
    run = pl.kernel(
        _gather5_body,
        out_type=jax.ShapeDtypeStruct((_BNP, _Ch), jnp.float32),
        mesh=mesh,
        scratch_types=[
            pltpu.VMEM((2, _K * _NPC), jnp.int32),
            pltpu.VMEM((2, _K, _NPC, _Ch), jnp.float32),
            pltpu.VMEM((_NPC, _Ch), jnp.float32),
            pltpu.SemaphoreType.DMA,
        ],
    )
    return run(table, idxs)


def _mk_idx(ni):
    ks = jnp.arange(_K, dtype=jnp.int32)[None, :]
    fused = (ks + 1) * _BN + ni
    npad = _BNP - _BN
    pad_rows = (jnp.arange(npad * _K, dtype=jnp.int32).reshape(npad, _K)
                * 79) % _BN
    fused = jnp.concatenate([fused, (ks + 1) * _BN + pad_rows], axis=0)
    return (fused.reshape(_TOTCH, _NPC, _K)
            .transpose(0, 2, 1).reshape(_TOTCH, _K * _NPC))


def kernel(features, neighbors_index, W1, b1, W2, b2, W3, b3, W4):
    del b1, b2, b3
    flat = features.reshape(_BN, _C)
    ni = neighbors_index.reshape(_BN, _K).astype(jnp.int32)

    w1t = W1.T
    m1 = jnp.concatenate(
        [w1t[_C * j:_C * (j + 1)] for j in range(_NSLOT)] + [W3.T], axis=1)
    w2t = W2.T
    m2 = jnp.concatenate(
        [w2t[_Ch * j:_Ch * (j + 1)] for j in range(_NSLOT)], axis=1)
    w4b = jnp.broadcast_to(W4, (8, 128))

    idx = _mk_idx(ni)

    table1 = _mm_slots(flat, m1)
    xpre1 = _gather5(table1, idx)
    mean1, rstd1 = _stats2(xpre1, table1)
    table2 = _norm_mm_slots(xpre1, table1, mean1, rstd1, m2)
    xpre2 = _gather5(table2, idx)
    mean2, rstd2 = _stats2(xpre2, table2)
    meanr, rstdr = _stats_slot(table1, _NSLOT)
    out8 = _final(xpre2, table2, table1, mean2, rstd2, meanr, rstdr, w4b)
    return out8[:, 0:1].reshape(_B, _N, 1)


# speedup vs baseline: 2.6142x; 1.3370x over previous
import functools

import jax
import jax.numpy as jnp
from jax import lax
from jax.experimental import pallas as pl
from jax.experimental.pallas import tpu as pltpu
from jax.experimental.pallas import tpu_sc as plsc

_B, _N, _K, _C, _Ch = 4, 10000, 5, 256, 128
_BN = _B * _N
_NSLOT = _K + 1
_NC, _NS = 2, 16
_NW = _NC * _NS
_BNP = 40960
_NPC = 64
_TOTCH = _BNP // _NPC
_NCH0 = 20
_NCH1 = (_TOTCH - _NS * _NCH0) // _NS
_RT = 2000
_NRT = _N // _RT
_NRTG = _BN // _RT

_PREC = jax.lax.Precision.HIGHEST


def _dot(a, b):
    return jax.lax.dot_general(a, b, (((1,), (0,)), ((), ())),
                               precision=_PREC,
                               preferred_element_type=jnp.float32)


def _lrelu(x):
    return jnp.where(x >= 0, x, 0.01 * x)


def _mm_body(x_ref, m_ref, *o_refs):
    r = _dot(x_ref[...], m_ref[...])
    for j, o in enumerate(o_refs):
        o[...] = r[:, j * _Ch:(j + 1) * _Ch]


def _mm_multi(x, m):
    cin = x.shape[1]
    nslots = m.shape[1] // _Ch
    return pl.pallas_call(
        _mm_body,
        grid=(_NRTG,),
        in_specs=[pl.BlockSpec((_RT, cin), lambda i: (i, 0)),
                  pl.BlockSpec((cin, nslots * _Ch), lambda i: (0, 0))],
        out_specs=[pl.BlockSpec((_RT, _Ch), lambda i: (i, 0))] * nslots,
        out_shape=[jax.ShapeDtypeStruct((_BN, _Ch), jnp.float32)] * nslots,
    )(x, m)


def _norm_mm_body(xp_ref, self_ref, mean_ref, rstd_ref, m_ref, *o_refs):
    x = (xp_ref[...] + self_ref[...] - mean_ref[0, 0:1, :]) * rstd_ref[0, 0:1, :]
    r = _dot(_lrelu(x), m_ref[...])
    for j, o in enumerate(o_refs):
        o[...] = r[:, j * _Ch:(j + 1) * _Ch]


def _norm_mm_multi(xp, selft, mean, rstd, m):
    nslots = m.shape[1] // _Ch
    return pl.pallas_call(
        _norm_mm_body,
        grid=(_NRTG,),
        in_specs=[pl.BlockSpec((_RT, _Ch), lambda i: (i, 0)),
                  pl.BlockSpec((_RT, _Ch), lambda i: (i, 0)),
                  pl.BlockSpec((1, 8, 128), lambda i: (i // _NRT, 0, 0)),
                  pl.BlockSpec((1, 8, 128), lambda i: (i // _NRT, 0, 0)),
                  pl.BlockSpec((_Ch, nslots * _Ch), lambda i: (0, 0))],
        out_specs=[pl.BlockSpec((_RT, _Ch), lambda i: (i, 0))] * nslots,
        out_shape=[jax.ShapeDtypeStruct((_BN, _Ch), jnp.float32)] * nslots,
    )(xp, selft, mean, rstd, m)


def _stats2_body(xp_ref, self_ref, mean_ref, rstd_ref):
    x = xp_ref[...] + self_ref[...]
    m = jnp.mean(x, axis=0, keepdims=True)
    v = jnp.mean((x - m) * (x - m), axis=0, keepdims=True)
    mean_ref[...] = jnp.broadcast_to(m[None], (1, 8, 128))
    rstd_ref[...] = jnp.broadcast_to(jax.lax.rsqrt(v + 1e-5)[None], (1, 8, 128))


def _stats2(xp, selft):
    return pl.pallas_call(
        _stats2_body,
        grid=(_B,),
        in_specs=[pl.BlockSpec((_N, _Ch), lambda b: (b, 0)),
                  pl.BlockSpec((_N, _Ch), lambda b: (b, 0))],
        out_specs=[pl.BlockSpec((1, 8, 128), lambda b: (b, 0, 0))] * 2,
        out_shape=[jax.ShapeDtypeStruct((_B, 8, 128), jnp.float32)] * 2,
    )(xp, selft)


def _stats3_body(xp_ref, self_ref, r_ref, mean_ref, rstd_ref, meanr_ref,
                 rstdr_ref):
    x = xp_ref[...] + self_ref[...]
    m = jnp.mean(x, axis=0, keepdims=True)
    v = jnp.mean((x - m) * (x - m), axis=0, keepdims=True)
    mean_ref[...] = jnp.broadcast_to(m[None], (1, 8, 128))
    rstd_ref[...] = jnp.broadcast_to(jax.lax.rsqrt(v + 1e-5)[None], (1, 8, 128))
    r = r_ref[...]
    mr = jnp.mean(r, axis=0, keepdims=True)
    vr = jnp.mean((r - mr) * (r - mr), axis=0, keepdims=True)
    meanr_ref[...] = jnp.broadcast_to(mr[None], (1, 8, 128))
    rstdr_ref[...] = jnp.broadcast_to(jax.lax.rsqrt(vr + 1e-5)[None],
                                      (1, 8, 128))


def _stats3(xp, selft, resid):
    return pl.pallas_call(
        _stats3_body,
        grid=(_B,),
        in_specs=[pl.BlockSpec((_N, _Ch), lambda b: (b, 0))] * 3,
        out_specs=[pl.BlockSpec((1, 8, 128), lambda b: (b, 0, 0))] * 4,
        out_shape=[jax.ShapeDtypeStruct((_B, 8, 128), jnp.float32)] * 4,
    )(xp, selft, resid)


def _final_body(x2_ref, self2_ref, r_ref, mean2_ref, rstd2_ref, meanr_ref,
                rstdr_ref, w4_ref, o_ref):
    x2n = ((x2_ref[...] + self2_ref[...] - mean2_ref[0, 0:1, :])
           * rstd2_ref[0, 0:1, :])
    rn = (r_ref[...] - meanr_ref[0, 0:1, :]) * rstdr_ref[0, 0:1, :]
    y = _lrelu(x2n + rn)
    s = jnp.sum(y * w4_ref[0:1, :], axis=1, keepdims=True)
    o_ref[...] = jnp.broadcast_to(s, (_RT, 8))


def _final(x2, self2, resid, mean2, rstd2, meanr, rstdr, w4b):
    return pl.pallas_call(
        _final_body,
        grid=(_NRTG,),
        in_specs=[pl.BlockSpec((_RT, _Ch), lambda i: (i, 0)),
                  pl.BlockSpec((_RT, _Ch), lambda i: (i, 0)),
                  pl.BlockSpec((_RT, _Ch), lambda i: (i, 0)),
                  pl.BlockSpec((1, 8, 128), lambda i: (i // _NRT, 0, 0)),
                  pl.BlockSpec((1, 8, 128), lambda i: (i // _NRT, 0, 0)),
                  pl.BlockSpec((1, 8, 128), lambda i: (i // _NRT, 0, 0)),
                  pl.BlockSpec((1, 8, 128), lambda i: (i // _NRT, 0, 0)),
                  pl.BlockSpec((8, 128), lambda i: (0, 0))],
        out_specs=pl.BlockSpec((_RT, 8), lambda i: (i, 0)),
        out_shape=jax.ShapeDtypeStruct((_BN, 8), jnp.float32),
    )(x2, self2, resid, mean2, rstd2, meanr, rstdr, w4b)


def _gather5_body(t0, t1, t2, t3, t4, idxs, out, idx_v, rows_v, acc_v,
                  gsem, osem):
    tables = (t0, t1, t2, t3, t4)
    cid = lax.axis_index("c")
    sid = lax.axis_index("s")
    start = jnp.where(cid == 0, sid * _NCH0, _NS * _NCH0 + sid * _NCH1)
    cnt = jnp.where(cid == 0, _NCH0, _NCH1)

    def gathers(buf):
        return [
            pltpu.make_async_copy(
                tables[k].at[idx_v.at[buf].at[pl.ds(k * _NPC, _NPC)]],
                rows_v.at[buf].at[k], gsem)
            for k in range(_K)
        ]

    def out_copy(ci, buf):
        return pltpu.make_async_copy(
            acc_v.at[buf], out.at[pl.ds((start + ci) * _NPC, _NPC)], osem)

    def fire(ci, buf):
        pltpu.sync_copy(idxs.at[start + ci], idx_v.at[buf])
        for cp in gathers(buf):
            cp.start()

    def chunk(ci, carry):
        p = lax.rem(ci, 2)

        @pl.when(ci < cnt - 1)
        def _():
            fire(ci + 1, 1 - p)

        for cp in gathers(p):
            cp.wait()

        @pl.when(ci >= 2)
        def _():
            out_copy(ci - 2, p).wait()

        def node(i, c2):
            for c8 in range(_Ch // 16):
                sl = pl.ds(c8 * 16, 16)
                acc_v[p, i, sl] = ((rows_v[p, 0, i, sl] + rows_v[p, 1, i, sl])
                                   + (rows_v[p, 2, i, sl] + rows_v[p, 3, i, sl])
                                   + rows_v[p, 4, i, sl])
            return c2

        lax.fori_loop(0, _NPC, node, 0)
        out_copy(ci, p).start()
        return carry

    fire(0, 0)
    lax.fori_loop(0, cnt, chunk, 0)
    out_copy(cnt - 2, lax.rem(cnt - 2, 2)).wait()
    out_copy(cnt - 1, lax.rem(cnt - 1, 2)).wait()


def _gather5(tables, idxs):
    mesh = plsc.VectorSubcoreMesh(core_axis_name="c", subcore_axis_name="s",
                                  num_cores=_NC, num_subcores=_NS)
    run = pl.kernel(
        _gather5_body,
        out_type=jax.ShapeDtypeStruct((_BNP, _Ch), jnp.float32),
        mesh=mesh,
        scratch_types=[
            pltpu.VMEM((2, _K * _NPC), jnp.int32),
            pltpu.VMEM((2, _K, _NPC, _Ch), jnp.float32),
            pltpu.VMEM((2, _NPC, _Ch), jnp.float32),
            pltpu.SemaphoreType.DMA,
            pltpu.SemaphoreType.DMA,
        ],
    )
    return run(*tables, idxs)


def _mk_idx(ni):
    npad = _BNP - _BN
    pad_rows = (jnp.arange(npad * _K, dtype=jnp.int32).reshape(npad, _K)
                * 79) % _BN
    fused = jnp.concatenate([ni, pad_rows], axis=0)
    return (fused.reshape(_TOTCH, _NPC, _K)
            .transpose(0, 2, 1).reshape(_TOTCH, _K * _NPC))


def kernel(features, neighbors_index, W1, b1, W2, b2, W3, b3, W4):
    del b1, b2, b3
    flat = features.reshape(_BN, _C)
    ni = neighbors_index.reshape(_BN, _K).astype(jnp.int32)

    w1t = W1.T
    m1 = jnp.concatenate(
        [w1t[_C * j:_C * (j + 1)] for j in range(_NSLOT)] + [W3.T], axis=1)
    w2t = W2.T
    m2 = jnp.concatenate(
        [w2t[_Ch * j:_Ch * (j + 1)] for j in range(_NSLOT)], axis=1)
    w4b = jnp.broadcast_to(W4, (8, 128))

    idx = _mk_idx(ni)

    outs1 = _mm_multi(flat, m1)
    self1, nbr1, resid = outs1[0], outs1[1:_NSLOT], outs1[_NSLOT]
    xpre1 = _gather5(nbr1, idx)
    mean1, rstd1 = _stats2(xpre1, self1)
    outs2 = _norm_mm_multi(xpre1, self1, mean1, rstd1, m2)
    self2, nbr2 = outs2[0], outs2[1:]
    xpre2 = _gather5(nbr2, idx)
    mean2, rstd2, meanr, rstdr = _stats3(xpre2, self2, resid)
    out8 = _final(xpre2, self2, resid, mean2, rstd2, meanr, rstdr, w4b)
    return out8[:, 0:1].reshape(_B, _N, 1)


# speedup vs baseline: 3.2282x; 1.2349x over previous
import functools

import jax
import jax.numpy as jnp
from jax import lax
from jax.experimental import pallas as pl
from jax.experimental.pallas import tpu as pltpu
from jax.experimental.pallas import tpu_sc as plsc

_B, _N, _K, _C, _Ch = 4, 10000, 5, 256, 128
_BN = _B * _N
_NSLOT = _K + 1
_NC, _NS = 2, 16
_NW = _NC * _NS
_BNP = 40960
_NPC = 64
_TOTCH = _BNP // _NPC
_NCH0 = 20
_NCH1 = (_TOTCH - _NS * _NCH0) // _NS
_RT = 2000
_NRT = _N // _RT
_NRTG = _BN // _RT

def _dot3(x, mh, ml):
    xh = x.astype(jnp.bfloat16)
    xl = (x - xh.astype(jnp.float32)).astype(jnp.bfloat16)
    d = functools.partial(jax.lax.dot_general,
                          dimension_numbers=(((1,), (0,)), ((), ())),
                          preferred_element_type=jnp.float32)
    return (d(xh, mh) + d(xh, ml)) + d(xl, mh)


def _split_bf16(m):
    mh = m.astype(jnp.bfloat16)
    ml = (m - mh.astype(jnp.float32)).astype(jnp.bfloat16)
    return mh, ml


def _lrelu(x):
    return jnp.where(x >= 0, x, 0.01 * x)


def _mm_body(x_ref, mh_ref, ml_ref, *o_refs):
    r = _dot3(x_ref[...], mh_ref[...], ml_ref[...])
    for j, o in enumerate(o_refs):
        o[...] = r[:, j * _Ch:(j + 1) * _Ch]


def _mm_multi(x, m):
    cin = x.shape[1]
    nslots = m.shape[1] // _Ch
    mh, ml = _split_bf16(m)
    return pl.pallas_call(
        _mm_body,
        grid=(_NRTG,),
        in_specs=[pl.BlockSpec((_RT, cin), lambda i: (i, 0)),
                  pl.BlockSpec((cin, nslots * _Ch), lambda i: (0, 0)),
                  pl.BlockSpec((cin, nslots * _Ch), lambda i: (0, 0))],
        out_specs=[pl.BlockSpec((_RT, _Ch), lambda i: (i, 0))] * nslots,
        out_shape=[jax.ShapeDtypeStruct((_BN, _Ch), jnp.float32)] * nslots,
    )(x, mh, ml)


def _norm_mm_body(xp_ref, self_ref, mean_ref, rstd_ref, mh_ref, ml_ref,
                  *o_refs):
    x = (xp_ref[...] + self_ref[...] - mean_ref[0, 0:1, :]) * rstd_ref[0, 0:1, :]
    r = _dot3(_lrelu(x), mh_ref[...], ml_ref[...])
    for j, o in enumerate(o_refs):
        o[...] = r[:, j * _Ch:(j + 1) * _Ch]


def _norm_mm_multi(xp, selft, mean, rstd, m):
    nslots = m.shape[1] // _Ch
    mh, ml = _split_bf16(m)
    return pl.pallas_call(
        _norm_mm_body,
        grid=(_NRTG,),
        in_specs=[pl.BlockSpec((_RT, _Ch), lambda i: (i, 0)),
                  pl.BlockSpec((_RT, _Ch), lambda i: (i, 0)),
                  pl.BlockSpec((1, 8, 128), lambda i: (i // _NRT, 0, 0)),
                  pl.BlockSpec((1, 8, 128), lambda i: (i // _NRT, 0, 0)),
                  pl.BlockSpec((_Ch, nslots * _Ch), lambda i: (0, 0)),
                  pl.BlockSpec((_Ch, nslots * _Ch), lambda i: (0, 0))],
        out_specs=[pl.BlockSpec((_RT, _Ch), lambda i: (i, 0))] * nslots,
        out_shape=[jax.ShapeDtypeStruct((_BN, _Ch), jnp.float32)] * nslots,
    )(xp, selft, mean, rstd, mh, ml)


def _stats2_body(xp_ref, self_ref, mean_ref, rstd_ref):
    x = xp_ref[...] + self_ref[...]
    m = jnp.mean(x, axis=0, keepdims=True)
    v = jnp.mean((x - m) * (x - m), axis=0, keepdims=True)
    mean_ref[...] = jnp.broadcast_to(m[None], (1, 8, 128))
    rstd_ref[...] = jnp.broadcast_to(jax.lax.rsqrt(v + 1e-5)[None], (1, 8, 128))


def _stats2(xp, selft):
    return pl.pallas_call(
        _stats2_body,
        grid=(_B,),
        in_specs=[pl.BlockSpec((_N, _Ch), lambda b: (b, 0)),
                  pl.BlockSpec((_N, _Ch), lambda b: (b, 0))],
        out_specs=[pl.BlockSpec((1, 8, 128), lambda b: (b, 0, 0))] * 2,
        out_shape=[jax.ShapeDtypeStruct((_B, 8, 128), jnp.float32)] * 2,
    )(xp, selft)


def _stats3_body(xp_ref, self_ref, r_ref, mean_ref, rstd_ref, meanr_ref,
                 rstdr_ref):
    x = xp_ref[...] + self_ref[...]
    m = jnp.mean(x, axis=0, keepdims=True)
    v = jnp.mean((x - m) * (x - m), axis=0, keepdims=True)
    mean_ref[...] = jnp.broadcast_to(m[None], (1, 8, 128))
    rstd_ref[...] = jnp.broadcast_to(jax.lax.rsqrt(v + 1e-5)[None], (1, 8, 128))
    r = r_ref[...]
    mr = jnp.mean(r, axis=0, keepdims=True)
    vr = jnp.mean((r - mr) * (r - mr), axis=0, keepdims=True)
    meanr_ref[...] = jnp.broadcast_to(mr[None], (1, 8, 128))
    rstdr_ref[...] = jnp.broadcast_to(jax.lax.rsqrt(vr + 1e-5)[None],
                                      (1, 8, 128))


def _stats3(xp, selft, resid):
    return pl.pallas_call(
        _stats3_body,
        grid=(_B,),
        in_specs=[pl.BlockSpec((_N, _Ch), lambda b: (b, 0))] * 3,
        out_specs=[pl.BlockSpec((1, 8, 128), lambda b: (b, 0, 0))] * 4,
        out_shape=[jax.ShapeDtypeStruct((_B, 8, 128), jnp.float32)] * 4,
    )(xp, selft, resid)


def _final_body(x2_ref, self2_ref, r_ref, mean2_ref, rstd2_ref, meanr_ref,
                rstdr_ref, w4_ref, o_ref):
    x2n = ((x2_ref[...] + self2_ref[...] - mean2_ref[0, 0:1, :])
           * rstd2_ref[0, 0:1, :])
    rn = (r_ref[...] - meanr_ref[0, 0:1, :]) * rstdr_ref[0, 0:1, :]
    y = _lrelu(x2n + rn)
    s = jnp.sum(y * w4_ref[0:1, :], axis=1, keepdims=True)
    o_ref[...] = jnp.broadcast_to(s, (_RT, 8))


def _final(x2, self2, resid, mean2, rstd2, meanr, rstdr, w4b):
    return pl.pallas_call(
        _final_body,
        grid=(_NRTG,),
        in_specs=[pl.BlockSpec((_RT, _Ch), lambda i: (i, 0)),
                  pl.BlockSpec((_RT, _Ch), lambda i: (i, 0)),
                  pl.BlockSpec((_RT, _Ch), lambda i: (i, 0)),
                  pl.BlockSpec((1, 8, 128), lambda i: (i // _NRT, 0, 0)),
                  pl.BlockSpec((1, 8, 128), lambda i: (i // _NRT, 0, 0)),
                  pl.BlockSpec((1, 8, 128), lambda i: (i // _NRT, 0, 0)),
                  pl.BlockSpec((1, 8, 128), lambda i: (i // _NRT, 0, 0)),
                  pl.BlockSpec((8, 128), lambda i: (0, 0))],
        out_specs=pl.BlockSpec((_RT, 8), lambda i: (i, 0)),
        out_shape=jax.ShapeDtypeStruct((_BN, 8), jnp.float32),
    )(x2, self2, resid, mean2, rstd2, meanr, rstdr, w4b)


def _gather5_body(t0, t1, t2, t3, t4, idxs, out, idx_v, rows_v, acc_v,
                  gsem, osem):
    tables = (t0, t1, t2, t3, t4)
    cid = lax.axis_index("c")
    sid = lax.axis_index("s")
    start = jnp.where(cid == 0, sid * _NCH0, _NS * _NCH0 + sid * _NCH1)
    cnt = jnp.where(cid == 0, _NCH0, _NCH1)

    def gathers(buf):
        return [
            pltpu.make_async_copy(
                tables[k].at[idx_v.at[buf].at[pl.ds(k * _NPC, _NPC)]],
                rows_v.at[buf].at[k], gsem)
            for k in range(_K)
        ]

    def out_copy(ci, buf):
        return pltpu.make_async_copy(
            acc_v.at[buf], out.at[pl.ds((start + ci) * _NPC, _NPC)], osem)

    def fire(ci, buf):
        pltpu.sync_copy(idxs.at[start + ci], idx_v.at[buf])
        for cp in gathers(buf):
            cp.start()

    def chunk(ci, carry):
        p = lax.rem(ci, 2)

        @pl.when(ci < cnt - 1)
        def _():
            fire(ci + 1, 1 - p)

        for cp in gathers(p):
            cp.wait()

        @pl.when(ci >= 2)
        def _():
            out_copy(ci - 2, p).wait()

        def node(i, c2):
            for c8 in range(_Ch // 16):
                sl = pl.ds(c8 * 16, 16)
                acc_v[p, i, sl] = ((rows_v[p, 0, i, sl] + rows_v[p, 1, i, sl])
                                   + (rows_v[p, 2, i, sl] + rows_v[p, 3, i, sl])
                                   + rows_v[p, 4, i, sl])
            return c2

        lax.fori_loop(0, _NPC, node, 0)
        out_copy(ci, p).start()
        return carry

    fire(0, 0)
    lax.fori_loop(0, cnt, chunk, 0)
    out_copy(cnt - 2, lax.rem(cnt - 2, 2)).wait()
    out_copy(cnt - 1, lax.rem(cnt - 1, 2)).wait()


def _gather5(tables, idxs):
    mesh = plsc.VectorSubcoreMesh(core_axis_name="c", subcore_axis_name="s",
                                  num_cores=_NC, num_subcores=_NS)
    run = pl.kernel(
        _gather5_body,
        out_type=jax.ShapeDtypeStruct((_BNP, _Ch), jnp.float32),
        mesh=mesh,
        scratch_types=[
            pltpu.VMEM((2, _K * _NPC), jnp.int32),
            pltpu.VMEM((2, _K, _NPC, _Ch), jnp.float32),
            pltpu.VMEM((2, _NPC, _Ch), jnp.float32),
            pltpu.SemaphoreType.DMA,
            pltpu.SemaphoreType.DMA,
        ],
    )
    return run(*tables, idxs)


def _mk_idx(ni):
    npad = _BNP - _BN
    pad_rows = (jnp.arange(npad * _K, dtype=jnp.int32).reshape(npad, _K)
                * 79) % _BN
    fused = jnp.concatenate([ni, pad_rows], axis=0)
    return (fused.reshape(_TOTCH, _NPC, _K)
            .transpose(0, 2, 1).reshape(_TOTCH, _K * _NPC))


def kernel(features, neighbors_index, W1, b1, W2, b2, W3, b3, W4):
    del b1, b2, b3
    flat = features.reshape(_BN, _C)
    ni = neighbors_index.reshape(_BN, _K).astype(jnp.int32)

    w1t = W1.T
    m1 = jnp.concatenate(
        [w1t[_C * j:_C * (j + 1)] for j in range(_NSLOT)] + [W3.T], axis=1)
    w2t = W2.T
    m2 = jnp.concatenate(
        [w2t[_Ch * j:_Ch * (j + 1)] for j in range(_NSLOT)], axis=1)
    w4b = jnp.broadcast_to(W4, (8, 128))

    idx = _mk_idx(ni)

    outs1 = _mm_multi(flat, m1)
    self1, nbr1, resid = outs1[0], outs1[1:_NSLOT], outs1[_NSLOT]
    xpre1 = _gather5(nbr1, idx)
    mean1, rstd1 = _stats2(xpre1, self1)
    outs2 = _norm_mm_multi(xpre1, self1, mean1, rstd1, m2)
    self2, nbr2 = outs2[0], outs2[1:]
    xpre2 = _gather5(nbr2, idx)
    mean2, rstd2, meanr, rstdr = _stats3(xpre2, self2, resid)
    out8 = _final(xpre2, self2, resid, mean2, rstd2, meanr, rstdr, w4b)
    return out8[:, 0:1].reshape(_B, _N, 1)


# speedup vs baseline: 3.2322x; 1.0012x over previous
import functools

import jax
import jax.numpy as jnp
from jax import lax
from jax.experimental import pallas as pl
from jax.experimental.pallas import tpu as pltpu
from jax.experimental.pallas import tpu_sc as plsc

_B, _N, _K, _C, _Ch = 4, 10000, 5, 256, 128
_BN = _B * _N
_NSLOT = _K + 1
_NC, _NS = 2, 16
_NW = _NC * _NS
_BNP = 40960
_NPC = 64
_TOTCH = _BNP // _NPC
_NCH0 = 20
_NCH1 = (_TOTCH - _NS * _NCH0) // _NS
_RT = 2000
_NRT = _N // _RT
_NRTG = _BN // _RT

def _dot3(x, mh, ml):
    xh = x.astype(jnp.bfloat16)
    xl = (x - xh.astype(jnp.float32)).astype(jnp.bfloat16)
    d = functools.partial(jax.lax.dot_general,
                          dimension_numbers=(((1,), (0,)), ((), ())),
                          preferred_element_type=jnp.float32)
    return (d(xh, mh) + d(xh, ml)) + d(xl, mh)


def _split_bf16(m):
    mh = m.astype(jnp.bfloat16)
    ml = (m - mh.astype(jnp.float32)).astype(jnp.bfloat16)
    return mh, ml


def _lrelu(x):
    return jnp.where(x >= 0, x, 0.01 * x)


def _store_slot(o, rj):
    o[...] = rj.astype(o.dtype)


def _mm_body(x_ref, mh_ref, ml_ref, *o_refs):
    r = _dot3(x_ref[...], mh_ref[...], ml_ref[...])
    for j, o in enumerate(o_refs):
        _store_slot(o, r[:, j * _Ch:(j + 1) * _Ch])


def _mm_multi(x, m, out_dtypes):
    cin = x.shape[1]
    nslots = m.shape[1] // _Ch
    mh, ml = _split_bf16(m)
    return pl.pallas_call(
        _mm_body,
        grid=(_NRTG,),
        in_specs=[pl.BlockSpec((_RT, cin), lambda i: (i, 0)),
                  pl.BlockSpec((cin, nslots * _Ch), lambda i: (0, 0)),
                  pl.BlockSpec((cin, nslots * _Ch), lambda i: (0, 0))],
        out_specs=[pl.BlockSpec((_RT, _Ch), lambda i: (i, 0))] * nslots,
        out_shape=[jax.ShapeDtypeStruct((_BN, _Ch), dt) for dt in out_dtypes],
    )(x, mh, ml)


def _norm_mm_body(xp_ref, self_ref, mean_ref, rstd_ref, mh_ref, ml_ref,
                  *o_refs):
    x = (xp_ref[...] + self_ref[...] - mean_ref[0, 0:1, :]) * rstd_ref[0, 0:1, :]
    r = _dot3(_lrelu(x), mh_ref[...], ml_ref[...])
    for j, o in enumerate(o_refs):
        _store_slot(o, r[:, j * _Ch:(j + 1) * _Ch])


def _norm_mm_multi(xp, selft, mean, rstd, m, out_dtypes):
    nslots = m.shape[1] // _Ch
    mh, ml = _split_bf16(m)
    return pl.pallas_call(
        _norm_mm_body,
        grid=(_NRTG,),
        in_specs=[pl.BlockSpec((_RT, _Ch), lambda i: (i, 0)),
                  pl.BlockSpec((_RT, _Ch), lambda i: (i, 0)),
                  pl.BlockSpec((1, 8, 128), lambda i: (i // _NRT, 0, 0)),
                  pl.BlockSpec((1, 8, 128), lambda i: (i // _NRT, 0, 0)),
                  pl.BlockSpec((_Ch, nslots * _Ch), lambda i: (0, 0)),
                  pl.BlockSpec((_Ch, nslots * _Ch), lambda i: (0, 0))],
        out_specs=[pl.BlockSpec((_RT, _Ch), lambda i: (i, 0))] * nslots,
        out_shape=[jax.ShapeDtypeStruct((_BN, _Ch), dt) for dt in out_dtypes],
    )(xp, selft, mean, rstd, mh, ml)


def _stats2_body(xp_ref, self_ref, mean_ref, rstd_ref):
    x = xp_ref[...] + self_ref[...]
    m = jnp.mean(x, axis=0, keepdims=True)
    v = jnp.mean((x - m) * (x - m), axis=0, keepdims=True)
    mean_ref[...] = jnp.broadcast_to(m[None], (1, 8, 128))
    rstd_ref[...] = jnp.broadcast_to(jax.lax.rsqrt(v + 1e-5)[None], (1, 8, 128))


def _stats2(xp, selft):
    return pl.pallas_call(
        _stats2_body,
        grid=(_B,),
        in_specs=[pl.BlockSpec((_N, _Ch), lambda b: (b, 0)),
                  pl.BlockSpec((_N, _Ch), lambda b: (b, 0))],
        out_specs=[pl.BlockSpec((1, 8, 128), lambda b: (b, 0, 0))] * 2,
        out_shape=[jax.ShapeDtypeStruct((_B, 8, 128), jnp.float32)] * 2,
    )(xp, selft)


def _stats3_body(xp_ref, self_ref, r_ref, mean_ref, rstd_ref, meanr_ref,
                 rstdr_ref):
    x = xp_ref[...] + self_ref[...]
    m = jnp.mean(x, axis=0, keepdims=True)
    v = jnp.mean((x - m) * (x - m), axis=0, keepdims=True)
    mean_ref[...] = jnp.broadcast_to(m[None], (1, 8, 128))
    rstd_ref[...] = jnp.broadcast_to(jax.lax.rsqrt(v + 1e-5)[None], (1, 8, 128))
    r = r_ref[...]
    mr = jnp.mean(r, axis=0, keepdims=True)
    vr = jnp.mean((r - mr) * (r - mr), axis=0, keepdims=True)
    meanr_ref[...] = jnp.broadcast_to(mr[None], (1, 8, 128))
    rstdr_ref[...] = jnp.broadcast_to(jax.lax.rsqrt(vr + 1e-5)[None],
                                      (1, 8, 128))


def _stats3(xp, selft, resid):
    return pl.pallas_call(
        _stats3_body,
        grid=(_B,),
        in_specs=[pl.BlockSpec((_N, _Ch), lambda b: (b, 0))] * 3,
        out_specs=[pl.BlockSpec((1, 8, 128), lambda b: (b, 0, 0))] * 4,
        out_shape=[jax.ShapeDtypeStruct((_B, 8, 128), jnp.float32)] * 4,
    )(xp, selft, resid)


def _final_body(x2_ref, self2_ref, r_ref, mean2_ref, rstd2_ref, meanr_ref,
                rstdr_ref, w4_ref, o_ref):
    x2n = ((x2_ref[...] + self2_ref[...] - mean2_ref[0, 0:1, :])
           * rstd2_ref[0, 0:1, :])
    rn = (r_ref[...] - meanr_ref[0, 0:1, :]) * rstdr_ref[0, 0:1, :]
    y = _lrelu(x2n + rn)
    s = jnp.sum(y * w4_ref[0:1, :], axis=1, keepdims=True)
    o_ref[...] = jnp.broadcast_to(s, (_RT, 8))


def _final(x2, self2, resid, mean2, rstd2, meanr, rstdr, w4b):
    return pl.pallas_call(
        _final_body,
        grid=(_NRTG,),
        in_specs=[pl.BlockSpec((_RT, _Ch), lambda i: (i, 0)),
                  pl.BlockSpec((_RT, _Ch), lambda i: (i, 0)),
                  pl.BlockSpec((_RT, _Ch), lambda i: (i, 0)),
                  pl.BlockSpec((1, 8, 128), lambda i: (i // _NRT, 0, 0)),
                  pl.BlockSpec((1, 8, 128), lambda i: (i // _NRT, 0, 0)),
                  pl.BlockSpec((1, 8, 128), lambda i: (i // _NRT, 0, 0)),
                  pl.BlockSpec((1, 8, 128), lambda i: (i // _NRT, 0, 0)),
                  pl.BlockSpec((8, 128), lambda i: (0, 0))],
        out_specs=pl.BlockSpec((_RT, 8), lambda i: (i, 0)),
        out_shape=jax.ShapeDtypeStruct((_BN, 8), jnp.float32),
    )(x2, self2, resid, mean2, rstd2, meanr, rstdr, w4b)


def _gather5_body(t0, t1, t2, t3, t4, idxs, out, idx_v, rows_v, acc_v,
                  gsem, osem):
    tables = (t0, t1, t2, t3, t4)
    cid = lax.axis_index("c")
    sid = lax.axis_index("s")
    start = jnp.where(cid == 0, sid * _NCH0, _NS * _NCH0 + sid * _NCH1)
    cnt = jnp.where(cid == 0, _NCH0, _NCH1)

    def gathers(buf):
        return [
            pltpu.make_async_copy(
                tables[k].at[idx_v.at[buf].at[pl.ds(k * _NPC, _NPC)]],
                rows_v.at[buf].at[k], gsem)
            for k in range(_K)
        ]

    def out_copy(ci, buf):
        return pltpu.make_async_copy(
            acc_v.at[buf], out.at[pl.ds((start + ci) * _NPC, _NPC)], osem)

    def fire(ci, buf):
        pltpu.sync_copy(idxs.at[start + ci], idx_v.at[buf])
        for cp in gathers(buf):
            cp.start()

    def chunk(ci, carry):
        p = lax.rem(ci, 2)

        @pl.when(ci < cnt - 1)
        def _():
            fire(ci + 1, 1 - p)

        for cp in gathers(p):
            cp.wait()

        @pl.when(ci >= 2)
        def _():
            out_copy(ci - 2, p).wait()

        def node(i, c2):
            for c8 in range(_Ch // 16):
                sl = pl.ds(c8 * 16, 16)
                acc_v[p, i, sl] = ((rows_v[p, 0, i, sl] + rows_v[p, 1, i, sl])
                                   + (rows_v[p, 2, i, sl] + rows_v[p, 3, i, sl])
                                   + rows_v[p, 4, i, sl])
            return c2

        lax.fori_loop(0, _NPC, node, 0)
        out_copy(ci, p).start()
        return carry

    fire(0, 0)
    lax.fori_loop(0, cnt, chunk, 0)
    out_copy(cnt - 2, lax.rem(cnt - 2, 2)).wait()
    out_copy(cnt - 1, lax.rem(cnt - 1, 2)).wait()


def _gather5(tables, idxs):
    mesh = plsc.VectorSubcoreMesh(core_axis_name="c", subcore_axis_name="s",
                                  num_cores=_NC, num_subcores=_NS)
    run = pl.kernel(
        _gather5_body,
        out_type=jax.ShapeDtypeStruct((_BNP, _Ch), jnp.float32),
        mesh=mesh,
        scratch_types=[
            pltpu.VMEM((2, _K * _NPC), jnp.int32),
            pltpu.VMEM((2, _K, _NPC, _Ch), jnp.float32),
            pltpu.VMEM((2, _NPC, _Ch), jnp.float32),
            pltpu.SemaphoreType.DMA,
            pltpu.SemaphoreType.DMA,
        ],
    )
    return run(*tables, idxs)


def _mk_idx(ni):
    npad = _BNP - _BN
    pad_rows = (jnp.arange(npad * _K, dtype=jnp.int32).reshape(npad, _K)
                * 79) % _BN
    fused = jnp.concatenate([ni, pad_rows], axis=0)
    return (fused.reshape(_TOTCH, _NPC, _K)
            .transpose(0, 2, 1).reshape(_TOTCH, _K * _NPC))


def kernel(features, neighbors_index, W1, b1, W2, b2, W3, b3, W4):
    del b1, b2, b3
    flat = features.reshape(_BN, _C)
    ni = neighbors_index.reshape(_BN, _K).astype(jnp.int32)

    w1t = W1.T
    m1 = jnp.concatenate(
        [w1t[_C * j:_C * (j + 1)] for j in range(_NSLOT)] + [W3.T], axis=1)
    w2t = W2.T
    m2 = jnp.concatenate(
        [w2t[_Ch * j:_Ch * (j + 1)] for j in range(_NSLOT)], axis=1)
    w4b = jnp.broadcast_to(W4, (8, 128))

    idx = _mk_idx(ni)

    bt = jnp.float32
    f32 = jnp.float32
    outs1 = _mm_multi(flat, m1, [f32] + [bt] * _K + [f32])
    self1, nbr1, resid = outs1[0], outs1[1:_NSLOT], outs1[_NSLOT]
    xpre1 = _gather5(nbr1, idx)
    mean1, rstd1 = _stats2(xpre1, self1)
    outs2 = _norm_mm_multi(xpre1, self1, mean1, rstd1, m2, [f32] + [bt] * _K)
    self2, nbr2 = outs2[0], outs2[1:]
    xpre2 = _gather5(nbr2, idx)
    mean2, rstd2, meanr, rstdr = _stats3(xpre2, self2, resid)
    out8 = _final(xpre2, self2, resid, mean2, rstd2, meanr, rstdr, w4b)
    return out8[:, 0:1].reshape(_B, _N, 1)
